# baseline (speedup 1.0000x reference)
import jax
import jax.numpy as jnp
from jax.experimental import pallas as pl
from jax.experimental.pallas import tpu as pltpu

_ROWS = 16384
_COLS = 128
_CHUNK = 512
_NCHUNKS = _ROWS // _CHUNK
_WINDOW = 4
_NSLOTS = 12


def _copy_set_kernel(x_hbm, o_hbm, scratch, in_sems, out_sems):
    def in_copy(c):
        return pltpu.make_async_copy(
            x_hbm.at[pl.ds(c * _CHUNK, _CHUNK), :],
            scratch.at[c % _NSLOTS], in_sems.at[c % _NSLOTS])

    def out_copy(c):
        return pltpu.make_async_copy(
            scratch.at[c % _NSLOTS],
            o_hbm.at[pl.ds(c * _CHUNK, _CHUNK), :], out_sems.at[c % _NSLOTS])

    out_waited = [False] * _NCHUNKS

    for c in range(min(_WINDOW, _NCHUNKS)):
        in_copy(c).start()
    for c in range(_NCHUNKS):
        in_copy(c).wait()
        if c == 0:
            col = jax.lax.broadcasted_iota(jnp.int32, (1, _COLS), 1)
            scratch[0, 0:1, :] = jnp.where(col == 0, 3.0, scratch[0, 0:1, :])
        out_copy(c).start()
        nw = c + _WINDOW
        if nw < _NCHUNKS:
            prev = nw - _NSLOTS
            if prev >= 0:
                out_copy(prev).wait()
                out_waited[prev] = True
            in_copy(nw).start()
    for c in range(_NCHUNKS):
        if not out_waited[c]:
            out_copy(c).wait()


def kernel(x):
    return pl.pallas_call(
        _copy_set_kernel,
        in_specs=[pl.BlockSpec(memory_space=pl.ANY)],
        out_specs=pl.BlockSpec(memory_space=pl.ANY),
        out_shape=jax.ShapeDtypeStruct((_ROWS, _COLS), jnp.float32),
        scratch_shapes=[
            pltpu.VMEM((_NSLOTS, _CHUNK, _COLS), jnp.float32),
            pltpu.SemaphoreType.DMA((_NSLOTS,)),
            pltpu.SemaphoreType.DMA((_NSLOTS,)),
        ],
    )(x)


# speedup vs baseline: 1.3381x; 1.3381x over previous
import jax
import jax.numpy as jnp
from jax.experimental import pallas as pl
from jax.experimental.pallas import tpu as pltpu

_ROWS = 16384
_COLS = 128
_CHUNK = 2048
_NCHUNKS = _ROWS // _CHUNK
_WINDOW = 2
_NSLOTS = 8


def _copy_set_kernel(x_hbm, o_hbm, scratch, in_sems, out_sems):
    def in_copy(c):
        return pltpu.make_async_copy(
            x_hbm.at[pl.ds(c * _CHUNK, _CHUNK), :],
            scratch.at[c % _NSLOTS], in_sems.at[c % _NSLOTS])

    def out_copy(c):
        return pltpu.make_async_copy(
            scratch.at[c % _NSLOTS],
            o_hbm.at[pl.ds(c * _CHUNK, _CHUNK), :], out_sems.at[c % _NSLOTS])

    out_waited = [False] * _NCHUNKS

    for c in range(min(_WINDOW, _NCHUNKS)):
        in_copy(c).start()
    for c in range(_NCHUNKS):
        in_copy(c).wait()
        if c == 0:
            col = jax.lax.broadcasted_iota(jnp.int32, (1, _COLS), 1)
            scratch[0, 0:1, :] = jnp.where(col == 0, 3.0, scratch[0, 0:1, :])
        out_copy(c).start()
        nw = c + _WINDOW
        if nw < _NCHUNKS:
            prev = nw - _NSLOTS
            if prev >= 0:
                out_copy(prev).wait()
                out_waited[prev] = True
            in_copy(nw).start()
    for c in range(_NCHUNKS):
        if not out_waited[c]:
            out_copy(c).wait()


def kernel(x):
    return pl.pallas_call(
        _copy_set_kernel,
        in_specs=[pl.BlockSpec(memory_space=pl.ANY)],
        out_specs=pl.BlockSpec(memory_space=pl.ANY),
        out_shape=jax.ShapeDtypeStruct((_ROWS, _COLS), jnp.float32),
        scratch_shapes=[
            pltpu.VMEM((_NSLOTS, _CHUNK, _COLS), jnp.float32),
            pltpu.SemaphoreType.DMA((_NSLOTS,)),
            pltpu.SemaphoreType.DMA((_NSLOTS,)),
        ],
    )(x)


# speedup vs baseline: 1.8467x; 1.3801x over previous
import jax
import jax.numpy as jnp
from jax.experimental import pallas as pl
from jax.experimental.pallas import tpu as pltpu

_ROWS = 16384
_COLS = 128
_SIZES = (256, 768, 1280, 1792, 2304, 2816, 3328, 3840)
_OFFS = tuple(sum(_SIZES[:i]) for i in range(len(_SIZES)))


def _copy_set_kernel(x_hbm, o_hbm, scratch, in_sems, out_sems):
    def in_copy(c):
        return pltpu.make_async_copy(
            x_hbm.at[pl.ds(_OFFS[c], _SIZES[c]), :],
            scratch.at[pl.ds(_OFFS[c], _SIZES[c]), :], in_sems.at[c])

    def out_copy(c):
        return pltpu.make_async_copy(
            scratch.at[pl.ds(_OFFS[c], _SIZES[c]), :],
            o_hbm.at[pl.ds(_OFFS[c], _SIZES[c]), :], out_sems.at[c])

    for c in range(len(_SIZES)):
        in_copy(c).start()
    for c in range(len(_SIZES)):
        in_copy(c).wait()
        if c == 0:
            col = jax.lax.broadcasted_iota(jnp.int32, (1, _COLS), 1)
            scratch[0:1, :] = jnp.where(col == 0, 3.0, scratch[0:1, :])
        out_copy(c).start()
    for c in range(len(_SIZES)):
        out_copy(c).wait()


def kernel(x):
    return pl.pallas_call(
        _copy_set_kernel,
        in_specs=[pl.BlockSpec(memory_space=pl.ANY)],
        out_specs=pl.BlockSpec(memory_space=pl.ANY),
        out_shape=jax.ShapeDtypeStruct((_ROWS, _COLS), jnp.float32),
        scratch_shapes=[
            pltpu.VMEM((_ROWS, _COLS), jnp.float32),
            pltpu.SemaphoreType.DMA((len(_SIZES),)),
            pltpu.SemaphoreType.DMA((len(_SIZES),)),
        ],
    )(x)
